# baseline (speedup 1.0000x reference)
import functools

import jax
import jax.numpy as jnp
from jax import lax
from jax.experimental import pallas as pl
from jax.experimental.pallas import tpu as pltpu
from jax.experimental.pallas import tpu_sc as plsc


def _iota16():
    return lax.iota(jnp.int32, 16)


def _make_gather_native(B4096, J200, V, D):
    mesh = plsc.VectorSubcoreMesh(core_axis_name="c", subcore_axis_name="s")
    n_chunks = J200 // 8

    @functools.partial(
        pl.kernel,
        mesh=mesh,
        compiler_params=pltpu.CompilerParams(
            use_tc_tiling_on_sc=False, needs_layout_passes=False),
        out_type=jax.ShapeDtypeStruct((J200, 2, 32, 8, 128), jnp.float32),
        scratch_types=[
            pltpu.VMEM((128, J200), jnp.int32),
            pltpu.VMEM((n_chunks, 1024), jnp.int32),
            pltpu.VMEM((1024, D), jnp.float32),
            pltpu.VMEM((1024, D), jnp.float32),
            pltpu.VMEM((8, 2, 8, 128), jnp.float32),
            pltpu.SemaphoreType.DMA,
            pltpu.SemaphoreType.DMA,
        ],
    )
    def gather(x_hbm, table_hbm, out5_hbm, x2v, idxT, rows_v, rows_b, och,
               gsem, gsem_b):
        wid = lax.axis_index("s") * 2 + lax.axis_index("c")
        pltpu.sync_copy(x_hbm.at[pl.ds(wid * 128, 128), :], x2v)

        def build(rg, carry):
            for s in range(8):
                for lg in range(8):
                    v = plsc.load_gather(
                        x2v,
                        [lg * 16 + _iota16(),
                         jnp.full((16,), 8 * rg + s, jnp.int32)])
                    idxT[rg, pl.ds(s * 128 + lg * 16, 16)] = v
            return carry

        lax.fori_loop(0, n_chunks, build, 0)

        def transpose_write(rg, rv):
            for s in range(8):
                for g in range(2):
                    for s2 in range(8):
                        vs = [plsc.load_gather(
                                  rv,
                                  [s * 128 + lg * 16 + _iota16(),
                                   jnp.full((16,), 8 * g + s2, jnp.int32)])
                              for lg in range(8)]
                        for lg in range(8):
                            och[s, g, s2, pl.ds(lg * 16, 16)] = vs[lg]
            pltpu.sync_copy(och, out5_hbm.at[pl.ds(8 * rg, 8), :, wid])

        def fire(rg, rv, sem):
            pltpu.async_copy(table_hbm.at[idxT.at[rg]], rv, sem)

        def drain(rv, sem):
            pltpu.make_async_copy(table_hbm.at[idxT.at[0]], rv, sem).wait()

        fire(0, rows_v, gsem)
        def pair(i, carry):
            fire(2 * i + 1, rows_b, gsem_b)
            drain(rows_v, gsem)
            transpose_write(2 * i, rows_v)
            fire(2 * i + 2, rows_v, gsem)
            drain(rows_b, gsem_b)
            transpose_write(2 * i + 1, rows_b)
            return carry

        lax.fori_loop(0, (n_chunks - 1) // 2, pair, 0)
        drain(rows_v, gsem)
        transpose_write(n_chunks - 1, rows_v)

    return gather


def kernel(x, table):
    V, D = table.shape
    B, J = x.shape
    out5 = _make_gather_native(B, J, V, D)(x.astype(jnp.int32), table)
    return jnp.transpose(out5, (2, 4, 0, 1, 3)).reshape(B, J, D)
